# Initial kernel scaffold; baseline (speedup 1.0000x reference)
import jax
import jax.numpy as jnp
from jax.experimental import pallas as pl


def kernel(grid, cos_h_all, sin_h_all, cos_w_all, sin_w_all):
    raise NotImplementedError("write your pallas kernel here")


# speedup vs baseline: 6.7216x; 6.7216x over previous
import jax
import jax.numpy as jnp
from jax import lax
from jax.experimental import pallas as pl
from jax.experimental.pallas import tpu as pltpu
from jax.experimental.pallas import tpu_sc as plsc

_B = 64
_T = 1024
_N = _B * _T
_NC = 2
_NS = 16
_NW = _NC * _NS
_CHUNK = _N // _NW
_GB = 128
_NG = _CHUNK // _GB
_D = 64


def _sc_body(h_hbm, w_hbm, sin_tab_hbm, cos_tab_hbm, sin_out_hbm, cos_out_hbm,
             h_v, w_v, idx_v, sin_rows, cos_rows, sem_s, sem_c):
    wid = lax.axis_index("s") * _NC + lax.axis_index("c")
    base = wid * _CHUNK
    pltpu.sync_copy(h_hbm.at[pl.ds(base, _CHUNK)], h_v)
    pltpu.sync_copy(w_hbm.at[pl.ds(base, _CHUNK)], w_v)

    def idx_body(j, carry):
        for k in range(_GB // 16):
            t0 = j * _GB + k * 16
            h = h_v[pl.ds(t0, 16)]
            w = w_v[pl.ds(t0, 16)]
            idx_v[j, pl.ds(k * 16, 16)] = h * 32 + w
        return carry

    lax.fori_loop(0, _NG, idx_body, 0)

    def gather_body(j, carry):
        cp_s = pltpu.async_copy(sin_tab_hbm.at[idx_v.at[j]], sin_rows, sem_s)
        cp_c = pltpu.async_copy(cos_tab_hbm.at[idx_v.at[j]], cos_rows, sem_c)
        cp_s.wait()
        pltpu.sync_copy(sin_rows, sin_out_hbm.at[pl.ds(base + j * _GB, _GB)])
        cp_c.wait()
        pltpu.sync_copy(cos_rows, cos_out_hbm.at[pl.ds(base + j * _GB, _GB)])
        return carry

    lax.fori_loop(0, _NG, gather_body, 0)


@jax.jit
def _rope_sc(grid, cos_h_all, sin_h_all, cos_w_all, sin_w_all):
    h_n, f = cos_h_all.shape
    w_n = cos_w_all.shape[0]
    ch = jnp.broadcast_to(cos_h_all[:, None, :], (h_n, w_n, f))
    cw = jnp.broadcast_to(cos_w_all[None, :, :], (h_n, w_n, f))
    sh = jnp.broadcast_to(sin_h_all[:, None, :], (h_n, w_n, f))
    sw = jnp.broadcast_to(sin_w_all[None, :, :], (h_n, w_n, f))
    cos_tab = jnp.concatenate([ch, cw, ch, cw], axis=-1).reshape(h_n * w_n, 4 * f)
    sin_tab = jnp.concatenate([sh, sw, sh, sw], axis=-1).reshape(h_n * w_n, 4 * f)
    h_flat = grid[..., 0].reshape(-1)
    w_flat = grid[..., 1].reshape(-1)

    mesh = plsc.VectorSubcoreMesh(core_axis_name="c", subcore_axis_name="s")
    ker = pl.kernel(
        _sc_body,
        out_type=[jax.ShapeDtypeStruct((_N, _D), jnp.float32),
                  jax.ShapeDtypeStruct((_N, _D), jnp.float32)],
        mesh=mesh,
        compiler_params=pltpu.CompilerParams(use_tc_tiling_on_sc=False),
        scratch_types=[
            pltpu.VMEM((_CHUNK,), jnp.int32),
            pltpu.VMEM((_CHUNK,), jnp.int32),
            pltpu.VMEM((_NG, _GB), jnp.int32),
            pltpu.VMEM((_GB, _D), jnp.float32),
            pltpu.VMEM((_GB, _D), jnp.float32),
            pltpu.SemaphoreType.DMA,
            pltpu.SemaphoreType.DMA,
        ],
    )
    sin_o, cos_o = ker(h_flat, w_flat, sin_tab, cos_tab)
    return sin_o.reshape(_B, _T, _D), cos_o.reshape(_B, _T, _D)


def kernel(grid, cos_h_all, sin_h_all, cos_w_all, sin_w_all):
    return _rope_sc(grid, cos_h_all, sin_h_all, cos_w_all, sin_w_all)


# speedup vs baseline: 6.7532x; 1.0047x over previous
import jax
import jax.numpy as jnp
from jax import lax
from jax.experimental import pallas as pl
from jax.experimental.pallas import tpu as pltpu
from jax.experimental.pallas import tpu_sc as plsc

_B = 64
_T = 1024
_N = _B * _T
_NC = 2
_NS = 16
_NW = _NC * _NS
_CHUNK = _N // _NW
_GB = 128
_NG = _CHUNK // _GB
_D = 64
_NB = 4
_NGRP = _NG // _NB


def _sc_body(h_hbm, w_hbm, sin_tab_hbm, cos_tab_hbm, sin_out_hbm, cos_out_hbm,
             h_v, w_v, idx_v, rows_s, rows_c, *sems):
    gsem_s = sems[0:_NB]
    gsem_c = sems[_NB:2 * _NB]
    wsem_s = sems[2 * _NB:3 * _NB]
    wsem_c = sems[3 * _NB:4 * _NB]
    wid = lax.axis_index("s") * _NC + lax.axis_index("c")
    base = wid * _CHUNK
    pltpu.sync_copy(h_hbm.at[pl.ds(base, _CHUNK)], h_v)
    pltpu.sync_copy(w_hbm.at[pl.ds(base, _CHUNK)], w_v)

    def idx_body(j, carry):
        for k in range(_GB // 16):
            t0 = j * _GB + k * 16
            h = h_v[pl.ds(t0, 16)]
            w = w_v[pl.ds(t0, 16)]
            idx_v[j, pl.ds(k * 16, 16)] = h * 32 + w
        return carry

    lax.fori_loop(0, _NG, idx_body, 0)

    def fire_gather(j, b):
        pltpu.async_copy(sin_tab_hbm.at[idx_v.at[j]], rows_s.at[b], gsem_s[b])
        pltpu.async_copy(cos_tab_hbm.at[idx_v.at[j]], rows_c.at[b], gsem_c[b])

    for b in range(_NB):
        fire_gather(b, b)

    def group_body(g, carry):
        for b in range(_NB):
            j = g * _NB + b
            dst_s = sin_out_hbm.at[pl.ds(base + j * _GB, _GB)]
            dst_c = cos_out_hbm.at[pl.ds(base + j * _GB, _GB)]
            pltpu.make_async_copy(sin_tab_hbm.at[idx_v.at[j]], rows_s.at[b],
                                  gsem_s[b]).wait()
            pltpu.make_async_copy(cos_tab_hbm.at[idx_v.at[j]], rows_c.at[b],
                                  gsem_c[b]).wait()
            cw_s = pltpu.async_copy(rows_s.at[b], dst_s, wsem_s[b])
            cw_c = pltpu.async_copy(rows_c.at[b], dst_c, wsem_c[b])

            @pl.when(g < _NGRP - 1)
            def _():
                cw_s.wait()
                cw_c.wait()
                fire_gather(j + _NB, b)

        return carry

    lax.fori_loop(0, _NGRP, group_body, 0)

    for b in range(_NB):
        j = (_NGRP - 1) * _NB + b
        pltpu.make_async_copy(rows_s.at[b],
                              sin_out_hbm.at[pl.ds(base + j * _GB, _GB)],
                              wsem_s[b]).wait()
        pltpu.make_async_copy(rows_c.at[b],
                              cos_out_hbm.at[pl.ds(base + j * _GB, _GB)],
                              wsem_c[b]).wait()


@jax.jit
def _rope_sc(grid, cos_h_all, sin_h_all, cos_w_all, sin_w_all):
    h_n, f = cos_h_all.shape
    w_n = cos_w_all.shape[0]
    ch = jnp.broadcast_to(cos_h_all[:, None, :], (h_n, w_n, f))
    cw = jnp.broadcast_to(cos_w_all[None, :, :], (h_n, w_n, f))
    sh = jnp.broadcast_to(sin_h_all[:, None, :], (h_n, w_n, f))
    sw = jnp.broadcast_to(sin_w_all[None, :, :], (h_n, w_n, f))
    cos_tab = jnp.concatenate([ch, cw, ch, cw], axis=-1).reshape(h_n * w_n, 4 * f)
    sin_tab = jnp.concatenate([sh, sw, sh, sw], axis=-1).reshape(h_n * w_n, 4 * f)
    h_flat = grid[..., 0].reshape(-1)
    w_flat = grid[..., 1].reshape(-1)

    mesh = plsc.VectorSubcoreMesh(core_axis_name="c", subcore_axis_name="s")
    ker = pl.kernel(
        _sc_body,
        out_type=[jax.ShapeDtypeStruct((_N, _D), jnp.float32),
                  jax.ShapeDtypeStruct((_N, _D), jnp.float32)],
        mesh=mesh,
        compiler_params=pltpu.CompilerParams(use_tc_tiling_on_sc=False),
        scratch_types=[
            pltpu.VMEM((_CHUNK,), jnp.int32),
            pltpu.VMEM((_CHUNK,), jnp.int32),
            pltpu.VMEM((_NG, _GB), jnp.int32),
            pltpu.VMEM((_NB, _GB, _D), jnp.float32),
            pltpu.VMEM((_NB, _GB, _D), jnp.float32),
        ] + [pltpu.SemaphoreType.DMA] * (4 * _NB),
    )
    sin_o, cos_o = ker(h_flat, w_flat, sin_tab, cos_tab)
    return sin_o.reshape(_B, _T, _D), cos_o.reshape(_B, _T, _D)


def kernel(grid, cos_h_all, sin_h_all, cos_w_all, sin_w_all):
    return _rope_sc(grid, cos_h_all, sin_h_all, cos_w_all, sin_w_all)


# speedup vs baseline: 7.0007x; 1.0366x over previous
import jax
import jax.numpy as jnp
from jax import lax
from jax.experimental import pallas as pl
from jax.experimental.pallas import tpu as pltpu
from jax.experimental.pallas import tpu_sc as plsc

_B = 64
_T = 1024
_N = _B * _T
_NC = 2
_NS = 16
_NW = _NC * _NS
_CHUNK = _N // _NW
_GB = 128
_NG = _CHUNK // _GB
_D = 128
_NB = 4
_NGRP = _NG // _NB


def _sc_body(h_hbm, w_hbm, tab_hbm, out_hbm, h_v, w_v, idx_v, rows, *sems):
    gsem = sems[:_NB]
    wsem = sems[_NB:]
    wid = lax.axis_index("s") * _NC + lax.axis_index("c")
    base = wid * _CHUNK
    pltpu.sync_copy(h_hbm.at[pl.ds(base, _CHUNK)], h_v)
    pltpu.sync_copy(w_hbm.at[pl.ds(base, _CHUNK)], w_v)

    def idx_body(j, carry):
        for k in range(_GB // 16):
            t0 = j * _GB + k * 16
            idx_v[j, pl.ds(k * 16, 16)] = h_v[pl.ds(t0, 16)] * 32 + w_v[pl.ds(t0, 16)]
        return carry

    lax.fori_loop(0, _NG, idx_body, 0)

    def fire_gather(j, b):
        pltpu.async_copy(tab_hbm.at[idx_v.at[j]], rows.at[b], gsem[b])

    for b in range(_NB):
        fire_gather(b, b)

    def group_body(g, carry):
        for b in range(_NB):
            j = g * _NB + b
            dst = out_hbm.at[pl.ds(base + j * _GB, _GB)]
            pltpu.make_async_copy(tab_hbm.at[idx_v.at[j]], rows.at[b],
                                  gsem[b]).wait()
            cw = pltpu.async_copy(rows.at[b], dst, wsem[b])

            @pl.when(g < _NGRP - 1)
            def _():
                cw.wait()
                fire_gather(j + _NB, b)

        return carry

    lax.fori_loop(0, _NGRP, group_body, 0)

    for b in range(_NB):
        j = (_NGRP - 1) * _NB + b
        pltpu.make_async_copy(rows.at[b],
                              out_hbm.at[pl.ds(base + j * _GB, _GB)],
                              wsem[b]).wait()


@jax.jit
def _rope_sc(grid, cos_h_all, sin_h_all, cos_w_all, sin_w_all):
    h_n, f = cos_h_all.shape
    w_n = cos_w_all.shape[0]
    ch = jnp.broadcast_to(cos_h_all[:, None, :], (h_n, w_n, f))
    cw = jnp.broadcast_to(cos_w_all[None, :, :], (h_n, w_n, f))
    sh = jnp.broadcast_to(sin_h_all[:, None, :], (h_n, w_n, f))
    sw = jnp.broadcast_to(sin_w_all[None, :, :], (h_n, w_n, f))
    tab = jnp.concatenate([sh, sw, sh, sw, ch, cw, ch, cw],
                          axis=-1).reshape(h_n * w_n, _D)
    h_flat = grid[..., 0].reshape(-1)
    w_flat = grid[..., 1].reshape(-1)

    mesh = plsc.VectorSubcoreMesh(core_axis_name="c", subcore_axis_name="s")
    ker = pl.kernel(
        _sc_body,
        out_type=jax.ShapeDtypeStruct((_N, _D), jnp.float32),
        mesh=mesh,
        scratch_types=[
            pltpu.VMEM((_CHUNK,), jnp.int32),
            pltpu.VMEM((_CHUNK,), jnp.int32),
            pltpu.VMEM((_NG, _GB), jnp.int32),
            pltpu.VMEM((_NB, _GB, _D), jnp.float32),
        ] + [pltpu.SemaphoreType.DMA] * (2 * _NB),
    )
    out = ker(h_flat, w_flat, tab)
    sin_o = out[:, :64].reshape(_B, _T, 64)
    cos_o = out[:, 64:].reshape(_B, _T, 64)
    return sin_o, cos_o


def kernel(grid, cos_h_all, sin_h_all, cos_w_all, sin_w_all):
    return _rope_sc(grid, cos_h_all, sin_h_all, cos_w_all, sin_w_all)
